# Initial kernel scaffold; baseline (speedup 1.0000x reference)
import jax
import jax.numpy as jnp
from jax.experimental import pallas as pl


def kernel(input, emb):
    raise NotImplementedError("write your pallas kernel here")


# speedup vs baseline: 1.7961x; 1.7961x over previous
import functools

import jax
import jax.numpy as jnp
from jax import lax
from jax.experimental import pallas as pl
from jax.experimental.pallas import tpu as pltpu
from jax.experimental.pallas import tpu_sc as plsc

EMBED_DIM = 64
_info = plsc.get_sparse_core_info()
_NC, _NS = _info.num_cores, _info.num_subcores
_NW = _NC * _NS

_CHUNK = 512


def _make_gather(B: int, V: int):
  b_per_w = B // _NW
  n_chunks = b_per_w // _CHUNK
  mesh = plsc.VectorSubcoreMesh(core_axis_name="c", subcore_axis_name="s")

  @functools.partial(
      pl.kernel,
      mesh=mesh,
      out_type=jax.ShapeDtypeStruct((B, EMBED_DIM), jnp.float32),
      scratch_types=[
          pltpu.VMEM((_CHUNK,), jnp.int32),
          pltpu.VMEM((_CHUNK, EMBED_DIM), jnp.float32),
          pltpu.SemaphoreType.DMA,
      ],
      compiler_params=pltpu.CompilerParams(use_tc_tiling_on_sc=False),
  )
  def gather_kernel(idx_hbm, table_hbm, out_hbm, idx_v, rows_v, sem):
    wid = lax.axis_index("s") * _NC + lax.axis_index("c")
    base = wid * b_per_w

    def body(i, carry):
      off = base + i * _CHUNK
      pltpu.sync_copy(idx_hbm.at[pl.ds(off, _CHUNK)], idx_v)
      pltpu.async_copy(table_hbm.at[idx_v], rows_v, sem).wait()
      pltpu.sync_copy(rows_v, out_hbm.at[pl.ds(off, _CHUNK)])
      return carry

    lax.fori_loop(0, n_chunks, body, 0)

  return gather_kernel


def kernel(input, emb):
  B0, B1 = input.shape
  V, D = emb.shape
  assert D == EMBED_DIM
  flat_idx = input.reshape(B0 * B1).astype(jnp.int32)
  out = _make_gather(B0 * B1, V)(flat_idx, emb)
  return out.reshape(B0, B1, D)


# speedup vs baseline: 1.8727x; 1.0426x over previous
import functools

import jax
import jax.numpy as jnp
from jax import lax
from jax.experimental import pallas as pl
from jax.experimental.pallas import tpu as pltpu
from jax.experimental.pallas import tpu_sc as plsc

EMBED_DIM = 64
_info = plsc.get_sparse_core_info()
_NC, _NS = _info.num_cores, _info.num_subcores
_NW = _NC * _NS

_CHUNK = 512


def _make_gather(B: int, V: int):
  b_per_w = B // _NW
  n_chunks = b_per_w // _CHUNK
  n_pairs = n_chunks // 2
  mesh = plsc.VectorSubcoreMesh(core_axis_name="c", subcore_axis_name="s")

  @functools.partial(
      pl.kernel,
      mesh=mesh,
      out_type=jax.ShapeDtypeStruct((B, EMBED_DIM), jnp.float32),
      scratch_types=[
          pltpu.VMEM((b_per_w,), jnp.int32),
          pltpu.VMEM((_CHUNK, EMBED_DIM), jnp.float32),
          pltpu.VMEM((_CHUNK, EMBED_DIM), jnp.float32),
          pltpu.SemaphoreType.DMA,
          pltpu.SemaphoreType.DMA,
          pltpu.SemaphoreType.DMA,
          pltpu.SemaphoreType.DMA,
      ],
      compiler_params=pltpu.CompilerParams(use_tc_tiling_on_sc=False),
  )
  def gather_kernel(idx_hbm, table_hbm, out_hbm, idx_v, rows0, rows1,
                    sg0, sg1, sw0, sw1):
    wid = lax.axis_index("s") * _NC + lax.axis_index("c")
    base = wid * b_per_w

    pltpu.sync_copy(idx_hbm.at[pl.ds(base, b_per_w)], idx_v)

    def gather(c, rows, sem):
      return pltpu.make_async_copy(
          table_hbm.at[idx_v.at[pl.ds(c * _CHUNK, _CHUNK)]], rows, sem)

    def write(c, rows, sem):
      return pltpu.make_async_copy(
          rows, out_hbm.at[pl.ds(base + c * _CHUNK, _CHUNK)], sem)

    gather(0, rows0, sg0).start()

    def pair(j, has_next):
      c0 = 2 * j
      gather(c0, rows0, sg0).wait()
      gather(c0 + 1, rows1, sg1).start()
      write(c0, rows0, sw0).start()
      gather(c0 + 1, rows1, sg1).wait()
      write(c0, rows0, sw0).wait()
      write(c0 + 1, rows1, sw1).start()
      if has_next:
        gather(c0 + 2, rows0, sg0).start()
      write(c0 + 1, rows1, sw1).wait()

    lax.fori_loop(0, n_pairs - 1, lambda j, c: (pair(j, True), c)[1], 0)
    pair(n_pairs - 1, False)

  return gather_kernel


def kernel(input, emb):
  B0, B1 = input.shape
  V, D = emb.shape
  assert D == EMBED_DIM
  flat_idx = input.reshape(B0 * B1).astype(jnp.int32)
  out = _make_gather(B0 * B1, V)(flat_idx, emb)
  return out.reshape(B0, B1, D)


# speedup vs baseline: 1.8758x; 1.0016x over previous
import functools

import jax
import jax.numpy as jnp
from jax import lax
from jax.experimental import pallas as pl
from jax.experimental.pallas import tpu as pltpu
from jax.experimental.pallas import tpu_sc as plsc

EMBED_DIM = 64
_info = plsc.get_sparse_core_info()
_NC, _NS = _info.num_cores, _info.num_subcores
_NW = _NC * _NS

_CHUNK = 256
_NBUF = 5


def _make_gather(B: int, V: int):
  b_per_w = B // _NW
  n_chunks = b_per_w // _CHUNK
  n_groups = n_chunks // _NBUF
  mesh = plsc.VectorSubcoreMesh(core_axis_name="c", subcore_axis_name="s")

  @functools.partial(
      pl.kernel,
      mesh=mesh,
      out_type=jax.ShapeDtypeStruct((B, EMBED_DIM), jnp.float32),
      scratch_types=[
          pltpu.VMEM((b_per_w,), jnp.int32),
          [pltpu.VMEM((_CHUNK, EMBED_DIM), jnp.float32) for _ in range(_NBUF)],
          [pltpu.SemaphoreType.DMA for _ in range(_NBUF)],
          [pltpu.SemaphoreType.DMA for _ in range(_NBUF)],
      ],
      compiler_params=pltpu.CompilerParams(use_tc_tiling_on_sc=False),
  )
  def gather_kernel(idx_hbm, table_hbm, out_hbm, idx_v, rows, sg, sw):
    wid = lax.axis_index("s") * _NC + lax.axis_index("c")
    base = wid * b_per_w

    pltpu.sync_copy(idx_hbm.at[pl.ds(base, b_per_w)], idx_v)

    def gather(c, b):
      return pltpu.make_async_copy(
          table_hbm.at[idx_v.at[pl.ds(c * _CHUNK, _CHUNK)]], rows[b], sg[b])

    def write(c, b):
      return pltpu.make_async_copy(
          rows[b], out_hbm.at[pl.ds(base + c * _CHUNK, _CHUNK)], sw[b])

    for b in range(_NBUF):
      gather(b, b).start()

    def group(j, refill):
      for b in range(_NBUF):
        c = j * _NBUF + b
        gather(c, b).wait()
        write(c, b).start()
        write(c, b).wait()
        if refill:
          gather(c + _NBUF, b).start()

    lax.fori_loop(0, n_groups - 1, lambda j, c: (group(j, True), c)[1], 0)
    group(n_groups - 1, False)

  return gather_kernel


def kernel(input, emb):
  B0, B1 = input.shape
  V, D = emb.shape
  assert D == EMBED_DIM
  flat_idx = input.reshape(B0 * B1).astype(jnp.int32)
  out = _make_gather(B0 * B1, V)(flat_idx, emb)
  return out.reshape(B0, B1, D)
